# Initial kernel scaffold; baseline (speedup 1.0000x reference)
import functools

import jax
import jax.numpy as jnp
from jax import lax
from jax.experimental import pallas as pl
from jax.experimental.pallas import tpu as pltpu
from jax.experimental.pallas import tpu_sc as plsc

F32 = jnp.float32
NC = 2
NS = 16
L = 16
NW = NC * NS


def _sc_mesh():
    return plsc.VectorSubcoreMesh(
        core_axis_name="c", subcore_axis_name="s", num_cores=NC, num_subcores=NS
    )


def _fill_1d(ref, total, val):
    vec = jnp.full((L,), val, F32)

    def body(i, _):
        ref[pl.ds(i * L, L)] = vec
        return _

    lax.fori_loop(0, total // L, body, None)


@functools.partial(jax.jit, static_argnums=(1,))
def _deg_pass(dst, n):
    e = dst.shape[0]
    epw = e // NW
    K = 2000
    nchunk = epw // K

    @functools.partial(
        pl.kernel,
        out_type=jax.ShapeDtypeStruct((NC, n), F32),
        mesh=_sc_mesh(),
        scratch_types=[
            pltpu.VMEM((K,), jnp.int32),
            pltpu.VMEM((K,), F32),
            pltpu.VMEM((K,), F32),
            pltpu.VMEM_SHARED((n,), F32),
        ],
    )
    def deg_k(dst_hbm, out_hbm, idx_v, ones_v, ini_v, acc_sh):
        cid = lax.axis_index("c")
        sid = lax.axis_index("s")
        wid = sid * NC + cid

        _fill_1d(ones_v, K, 1.0)

        @pl.when(sid == 0)
        def _():
            initval = jnp.where(cid == 0, 1.0, 0.0).astype(F32)
            _fill_1d(ini_v, K, initval)

            def cp(i, _):
                pltpu.sync_copy(ini_v, acc_sh.at[pl.ds(i * K, K)])
                return _

            lax.fori_loop(0, n // K, cp, None)

        plsc.subcore_barrier()

        base = wid * epw

        def chunk(i, _):
            pltpu.sync_copy(dst_hbm.at[pl.ds(base + i * K, K)], idx_v)
            pltpu.sync_copy(ones_v, acc_sh.at[idx_v], add=True)
            return _

        lax.fori_loop(0, nchunk, chunk, None)
        plsc.subcore_barrier()

        nwo = n // 1000

        @pl.when(sid < nwo)
        def _():
            pltpu.sync_copy(acc_sh.at[pl.ds(sid * 1000, 1000)], ini_v.at[pl.ds(0, 1000)])
            pltpu.sync_copy(ini_v.at[pl.ds(0, 1000)], out_hbm.at[cid, pl.ds(sid * 1000, 1000)])

    return deg_k(dst)


@functools.partial(jax.jit, static_argnums=(3, 4))
def _msg_pass(src, dst, y, n, w):
    e = src.shape[0]
    epw = e // NW
    K = 1000
    nchunk = epw // K
    rpw = n // NS

    @functools.partial(
        pl.kernel,
        out_type=jax.ShapeDtypeStruct((NC, n, w), F32),
        mesh=_sc_mesh(),
        scratch_types=[
            pltpu.VMEM((K,), jnp.int32),
            pltpu.VMEM((K,), jnp.int32),
            pltpu.VMEM((K, w), F32),
            pltpu.VMEM((rpw, w), F32),
            pltpu.VMEM_SHARED((n, w), F32),
            pltpu.SemaphoreType.DMA,
        ],
    )
    def msg_k(src_hbm, dst_hbm, y_hbm, out_hbm, sidx_v, didx_v, rows_v, stg_v, acc_sh, sem):
        cid = lax.axis_index("c")
        sid = lax.axis_index("s")
        wid = sid * NC + cid

        zvec = jnp.zeros((L,), F32)

        def zb(r, _):
            for j in range(w // L):
                stg_v[r, pl.ds(j * L, L)] = zvec
            return _

        lax.fori_loop(0, rpw, zb, None)
        pltpu.sync_copy(stg_v, acc_sh.at[pl.ds(sid * rpw, rpw)])
        plsc.subcore_barrier()

        base = wid * epw

        def chunk(i, _):
            pltpu.sync_copy(src_hbm.at[pl.ds(base + i * K, K)], sidx_v)
            pltpu.sync_copy(dst_hbm.at[pl.ds(base + i * K, K)], didx_v)
            pltpu.async_copy(y_hbm.at[sidx_v], rows_v, sem).wait()
            pltpu.sync_copy(rows_v, acc_sh.at[didx_v], add=True)
            return _

        lax.fori_loop(0, nchunk, chunk, None)
        plsc.subcore_barrier()

        pltpu.sync_copy(acc_sh.at[pl.ds(sid * rpw, rpw)], stg_v)
        pltpu.sync_copy(stg_v, out_hbm.at[cid, pl.ds(sid * rpw, rpw)])

    return msg_k(src, dst, y)


_BLK = 1000


def _stage1_body(deg0_ref, deg1_ref, x_ref, w1_ref, y1_ref, dinv_ref):
    deg = deg0_ref[...] + deg1_ref[...]
    dinv = lax.rsqrt(deg)
    xw = jnp.dot(x_ref[...], w1_ref[...], preferred_element_type=F32)
    y1_ref[...] = xw * dinv
    dinv_ref[...] = dinv


@jax.jit
def _tc_stage1(deg_parts, x, w1):
    n, d_in = x.shape
    d_hid = w1.shape[1]
    deg0 = deg_parts[0].reshape(n, 1)
    deg1 = deg_parts[1].reshape(n, 1)
    return pl.pallas_call(
        _stage1_body,
        grid=(n // _BLK,),
        in_specs=[
            pl.BlockSpec((_BLK, 1), lambda i: (i, 0)),
            pl.BlockSpec((_BLK, 1), lambda i: (i, 0)),
            pl.BlockSpec((_BLK, d_in), lambda i: (i, 0)),
            pl.BlockSpec((d_in, d_hid), lambda i: (0, 0)),
        ],
        out_specs=[
            pl.BlockSpec((_BLK, d_hid), lambda i: (i, 0)),
            pl.BlockSpec((_BLK, 1), lambda i: (i, 0)),
        ],
        out_shape=[
            jax.ShapeDtypeStruct((n, d_hid), F32),
            jax.ShapeDtypeStruct((n, 1), F32),
        ],
    )(deg0, deg1, x, w1)


def _stage2_body(a0_ref, a1_ref, y1_ref, dinv_ref, b1_ref, w2_ref, y2_ref):
    dinv = dinv_ref[...]
    h = (a0_ref[...] + a1_ref[...] + y1_ref[...]) * dinv + b1_ref[...]
    h = jnp.maximum(h, 0.0)
    y2_ref[...] = jnp.dot(h, w2_ref[...], preferred_element_type=F32) * dinv


@jax.jit
def _tc_stage2(p_parts, y1, dinv, b1r, w2p):
    n, d_hid = y1.shape
    wpad = w2p.shape[1]
    return pl.pallas_call(
        _stage2_body,
        grid=(n // _BLK,),
        in_specs=[
            pl.BlockSpec((_BLK, d_hid), lambda i: (i, 0)),
            pl.BlockSpec((_BLK, d_hid), lambda i: (i, 0)),
            pl.BlockSpec((_BLK, d_hid), lambda i: (i, 0)),
            pl.BlockSpec((_BLK, 1), lambda i: (i, 0)),
            pl.BlockSpec((1, d_hid), lambda i: (0, 0)),
            pl.BlockSpec((d_hid, wpad), lambda i: (0, 0)),
        ],
        out_specs=pl.BlockSpec((_BLK, wpad), lambda i: (i, 0)),
        out_shape=jax.ShapeDtypeStruct((n, wpad), F32),
    )(p_parts[0], p_parts[1], y1, dinv, b1r, w2p)


def _stage3_body(q0_ref, q1_ref, y2_ref, dinv_ref, b2_ref, out_ref):
    out_ref[...] = (
        (q0_ref[...] + q1_ref[...] + y2_ref[...]) * dinv_ref[...] + b2_ref[...]
    )


@jax.jit
def _tc_stage3(q_parts, y2p, dinv, b2p):
    n, wpad = y2p.shape
    return pl.pallas_call(
        _stage3_body,
        grid=(n // _BLK,),
        in_specs=[
            pl.BlockSpec((_BLK, wpad), lambda i: (i, 0)),
            pl.BlockSpec((_BLK, wpad), lambda i: (i, 0)),
            pl.BlockSpec((_BLK, wpad), lambda i: (i, 0)),
            pl.BlockSpec((_BLK, 1), lambda i: (i, 0)),
            pl.BlockSpec((1, wpad), lambda i: (0, 0)),
        ],
        out_specs=pl.BlockSpec((_BLK, wpad), lambda i: (i, 0)),
        out_shape=jax.ShapeDtypeStruct((n, wpad), F32),
    )(q_parts[0], q_parts[1], y2p, dinv, b2p)


def kernel(x, edge_index, W1, b1, W2, b2):
    n, _ = x.shape
    d_hid = W1.shape[1]
    d_out = W2.shape[1]
    wpad = 16

    src = edge_index[0]
    dst = edge_index[1]

    deg_parts = _deg_pass(dst, n)
    y1, dinv = _tc_stage1(deg_parts, x, W1)
    p_parts = _msg_pass(src, dst, y1, n, d_hid)

    b1r = b1.reshape(1, d_hid)
    w2p = jnp.zeros((d_hid, wpad), F32).at[:, :d_out].set(W2)
    b2p = jnp.zeros((1, wpad), F32).at[0, :d_out].set(b2)

    y2p = _tc_stage2(p_parts, y1, dinv, b1r, w2p)
    q_parts = _msg_pass(src, dst, y2p, n, wpad)
    out = _tc_stage3(q_parts, y2p, dinv, b2p)
    return out[:, :d_out]


# speedup vs baseline: 44.8068x; 44.8068x over previous
import functools

import jax
import jax.numpy as jnp
from jax import lax
from jax.experimental import pallas as pl
from jax.experimental.pallas import tpu as pltpu
from jax.experimental.pallas import tpu_sc as plsc

F32 = jnp.float32
NC = 2
NS = 16
L = 16
NW = NC * NS


def _sc_mesh():
    return plsc.VectorSubcoreMesh(
        core_axis_name="c", subcore_axis_name="s", num_cores=NC, num_subcores=NS
    )


def _fill_1d(ref, total, val):
    vec = jnp.full((L,), val, F32)

    def body(i, _):
        ref[pl.ds(i * L, L)] = vec
        return _

    lax.fori_loop(0, total // L, body, None)


@functools.partial(jax.jit, static_argnums=(1,))
def _deg_pass(dst, n):
    e = dst.shape[0]
    epw = e // NW
    K = 2000
    nchunk = epw // K

    @functools.partial(
        pl.kernel,
        out_type=jax.ShapeDtypeStruct((NC * n,), F32),
        mesh=_sc_mesh(),
        scratch_types=[
            pltpu.VMEM((K,), jnp.int32),
            pltpu.VMEM((K,), F32),
            pltpu.VMEM((K,), F32),
            pltpu.VMEM_SHARED((n,), F32),
        ],
    )
    def deg_k(dst_hbm, out_hbm, idx_v, ones_v, ini_v, acc_sh):
        cid = lax.axis_index("c")
        sid = lax.axis_index("s")
        wid = sid * NC + cid

        _fill_1d(ones_v, K, 1.0)

        @pl.when(sid == 0)
        def _():
            initval = jnp.where(cid == 0, 1.0, 0.0).astype(F32)
            _fill_1d(ini_v, K, initval)

            def cp(i, _):
                pltpu.sync_copy(ini_v, acc_sh.at[pl.ds(i * K, K)])
                return _

            lax.fori_loop(0, n // K, cp, None)

        plsc.subcore_barrier()

        base = wid * epw

        def chunk(i, _):
            pltpu.sync_copy(dst_hbm.at[pl.ds(base + i * K, K)], idx_v)
            pltpu.sync_copy(ones_v, acc_sh.at[idx_v], add=True)
            return _

        lax.fori_loop(0, nchunk, chunk, None)
        plsc.subcore_barrier()

        nwo = n // 1000

        @pl.when(sid < nwo)
        def _():
            pltpu.sync_copy(acc_sh.at[pl.ds(sid * 1000, 1000)], ini_v.at[pl.ds(0, 1000)])
            pltpu.sync_copy(
                ini_v.at[pl.ds(0, 1000)], out_hbm.at[pl.ds(cid * n + sid * 1000, 1000)]
            )

    return deg_k(dst).reshape(NC, n)


@functools.partial(jax.jit, static_argnums=(3, 4))
def _msg_pass(src, dst, y, n, w):
    e = src.shape[0]
    epw = e // NW
    K = 400
    nchunk = epw // K
    dst3 = dst.reshape(NW, nchunk, K)

    @functools.partial(
        pl.kernel,
        out_type=jax.ShapeDtypeStruct((NC, n, w), F32),
        mesh=_sc_mesh(),
        scratch_types=[
            pltpu.VMEM((epw,), jnp.int32),
            pltpu.VMEM((nchunk, K), jnp.int32),
            pltpu.VMEM((2, K, w), F32),
            pltpu.VMEM_SHARED((n, w), F32),
            pltpu.SemaphoreType.DMA,
            pltpu.SemaphoreType.DMA,
        ],
        compiler_params=pltpu.CompilerParams(use_tc_tiling_on_sc=False),
    )
    def msg_k(src_hbm, dst_hbm, y_hbm, out_hbm, sidx_v, didx_v, rows_v, acc_sh, sem0, sem1):
        cid = lax.axis_index("c")
        sid = lax.axis_index("s")
        wid = sid * NC + cid
        sems = [sem0, sem1]

        zvec = jnp.zeros((L,), F32)

        @pl.when(sid < n // 1000)
        def _():
            def zb(r, _):
                for j in range(w // L):
                    rows_v[0, r, pl.ds(j * L, L)] = zvec
                return _

            lax.fori_loop(0, K, zb, None)
            for off, sz in ((0, K), (K, K), (2 * K, 1000 - 2 * K)):
                pltpu.sync_copy(
                    rows_v.at[0, pl.ds(0, sz)],
                    acc_sh.at[pl.ds(sid * 1000 + off, sz)],
                )

        plsc.subcore_barrier()

        base = wid * epw
        pltpu.sync_copy(src_hbm.at[pl.ds(base, epw)], sidx_v)
        pltpu.sync_copy(dst_hbm.at[wid], didx_v)

        g = [None, None]
        g[0] = pltpu.async_copy(y_hbm.at[sidx_v.at[pl.ds(0, K)]], rows_v.at[0], sem0)
        for i in range(nchunk):
            s = i % 2
            o = 1 - s
            if i + 1 < nchunk:
                g[o] = pltpu.async_copy(
                    y_hbm.at[sidx_v.at[pl.ds((i + 1) * K, K)]], rows_v.at[o], sems[o]
                )
            g[s].wait()
            pltpu.sync_copy(rows_v.at[s], acc_sh.at[didx_v.at[i]], add=True)

        plsc.subcore_barrier()

        @pl.when(sid < n // 1000)
        def _():
            for off, sz in ((0, K), (K, K), (2 * K, 1000 - 2 * K)):
                pltpu.sync_copy(
                    acc_sh.at[pl.ds(sid * 1000 + off, sz)],
                    rows_v.at[0, pl.ds(0, sz)],
                )
                pltpu.sync_copy(
                    rows_v.at[0, pl.ds(0, sz)],
                    out_hbm.at[cid, pl.ds(sid * 1000 + off, sz)],
                )

    return msg_k(src, dst3, y)


_BLK = 1000


def _stage1_body(deg0_ref, deg1_ref, x_ref, w1_ref, y1_ref, dinv_ref):
    deg = deg0_ref[...] + deg1_ref[...]
    dinv = lax.rsqrt(deg)
    xw = jnp.dot(x_ref[...], w1_ref[...], preferred_element_type=F32)
    y1_ref[...] = xw * dinv
    dinv_ref[...] = dinv


@jax.jit
def _tc_stage1(deg_parts, x, w1):
    n, d_in = x.shape
    d_hid = w1.shape[1]
    deg0 = deg_parts[0].reshape(n, 1)
    deg1 = deg_parts[1].reshape(n, 1)
    return pl.pallas_call(
        _stage1_body,
        grid=(n // _BLK,),
        in_specs=[
            pl.BlockSpec((_BLK, 1), lambda i: (i, 0)),
            pl.BlockSpec((_BLK, 1), lambda i: (i, 0)),
            pl.BlockSpec((_BLK, d_in), lambda i: (i, 0)),
            pl.BlockSpec((d_in, d_hid), lambda i: (0, 0)),
        ],
        out_specs=[
            pl.BlockSpec((_BLK, d_hid), lambda i: (i, 0)),
            pl.BlockSpec((_BLK, 1), lambda i: (i, 0)),
        ],
        out_shape=[
            jax.ShapeDtypeStruct((n, d_hid), F32),
            jax.ShapeDtypeStruct((n, 1), F32),
        ],
    )(deg0, deg1, x, w1)


def _stage2_body(a0_ref, a1_ref, y1_ref, dinv_ref, b1_ref, w2_ref, y2_ref):
    dinv = dinv_ref[...]
    h = (a0_ref[...] + a1_ref[...] + y1_ref[...]) * dinv + b1_ref[...]
    h = jnp.maximum(h, 0.0)
    y2_ref[...] = jnp.dot(h, w2_ref[...], preferred_element_type=F32) * dinv


@jax.jit
def _tc_stage2(p_parts, y1, dinv, b1r, w2p):
    n, d_hid = y1.shape
    wpad = w2p.shape[1]
    return pl.pallas_call(
        _stage2_body,
        grid=(n // _BLK,),
        in_specs=[
            pl.BlockSpec((_BLK, d_hid), lambda i: (i, 0)),
            pl.BlockSpec((_BLK, d_hid), lambda i: (i, 0)),
            pl.BlockSpec((_BLK, d_hid), lambda i: (i, 0)),
            pl.BlockSpec((_BLK, 1), lambda i: (i, 0)),
            pl.BlockSpec((1, d_hid), lambda i: (0, 0)),
            pl.BlockSpec((d_hid, wpad), lambda i: (0, 0)),
        ],
        out_specs=pl.BlockSpec((_BLK, wpad), lambda i: (i, 0)),
        out_shape=jax.ShapeDtypeStruct((n, wpad), F32),
    )(p_parts[0], p_parts[1], y1, dinv, b1r, w2p)


def _stage3_body(q0_ref, q1_ref, y2_ref, dinv_ref, b2_ref, out_ref):
    out_ref[...] = (
        (q0_ref[...] + q1_ref[...] + y2_ref[...]) * dinv_ref[...] + b2_ref[...]
    )


@jax.jit
def _tc_stage3(q_parts, y2p, dinv, b2p):
    n, wpad = y2p.shape
    return pl.pallas_call(
        _stage3_body,
        grid=(n // _BLK,),
        in_specs=[
            pl.BlockSpec((_BLK, wpad), lambda i: (i, 0)),
            pl.BlockSpec((_BLK, wpad), lambda i: (i, 0)),
            pl.BlockSpec((_BLK, wpad), lambda i: (i, 0)),
            pl.BlockSpec((_BLK, 1), lambda i: (i, 0)),
            pl.BlockSpec((1, wpad), lambda i: (0, 0)),
        ],
        out_specs=pl.BlockSpec((_BLK, wpad), lambda i: (i, 0)),
        out_shape=jax.ShapeDtypeStruct((n, wpad), F32),
    )(q_parts[0], q_parts[1], y2p, dinv, b2p)


def kernel(x, edge_index, W1, b1, W2, b2):
    n, _ = x.shape
    d_hid = W1.shape[1]
    d_out = W2.shape[1]
    wpad = 16

    src = edge_index[0]
    dst = edge_index[1]

    deg_parts = _deg_pass(dst, n)
    y1, dinv = _tc_stage1(deg_parts, x, W1)
    p_parts = _msg_pass(src, dst, y1, n, d_hid)

    b1r = b1.reshape(1, d_hid)
    w2p = jnp.zeros((d_hid, wpad), F32).at[:, :d_out].set(W2)
    b2p = jnp.zeros((1, wpad), F32).at[0, :d_out].set(b2)

    y2p = _tc_stage2(p_parts, y1, dinv, b1r, w2p)
    q_parts = _msg_pass(src, dst, y2p, n, wpad)
    out = _tc_stage3(q_parts, y2p, dinv, b2p)
    return out[:, :d_out]


# speedup vs baseline: 47.9788x; 1.0708x over previous
import functools

import jax
import jax.numpy as jnp
from jax import lax
from jax.experimental import pallas as pl
from jax.experimental.pallas import tpu as pltpu
from jax.experimental.pallas import tpu_sc as plsc

F32 = jnp.float32
NC = 2
NS = 16
L = 16
NW = NC * NS


def _sc_mesh():
    return plsc.VectorSubcoreMesh(
        core_axis_name="c", subcore_axis_name="s", num_cores=NC, num_subcores=NS
    )


def _fill_1d(ref, total, val):
    vec = jnp.full((L,), val, F32)

    def body(i, _):
        ref[pl.ds(i * L, L)] = vec
        return _

    lax.fori_loop(0, total // L, body, None)


@functools.partial(jax.jit, static_argnums=(1,))
def _deg_pass(dst, n):
    e = dst.shape[0]
    epw = e // NW
    K = 2000
    nchunk = epw // K

    @functools.partial(
        pl.kernel,
        out_type=jax.ShapeDtypeStruct((NC * n,), F32),
        mesh=_sc_mesh(),
        scratch_types=[
            pltpu.VMEM((K,), jnp.int32),
            pltpu.VMEM((K,), F32),
            pltpu.VMEM((K,), F32),
            pltpu.VMEM_SHARED((n,), F32),
        ],
    )
    def deg_k(dst_hbm, out_hbm, idx_v, ones_v, ini_v, acc_sh):
        cid = lax.axis_index("c")
        sid = lax.axis_index("s")
        wid = sid * NC + cid

        _fill_1d(ones_v, K, 1.0)

        @pl.when(sid == 0)
        def _():
            initval = jnp.where(cid == 0, 1.0, 0.0).astype(F32)
            _fill_1d(ini_v, K, initval)

            def cp(i, _):
                pltpu.sync_copy(ini_v, acc_sh.at[pl.ds(i * K, K)])
                return _

            lax.fori_loop(0, n // K, cp, None)

        plsc.subcore_barrier()

        base = wid * epw

        def chunk(i, _):
            pltpu.sync_copy(dst_hbm.at[pl.ds(base + i * K, K)], idx_v)
            pltpu.sync_copy(ones_v, acc_sh.at[idx_v], add=True)
            return _

        lax.fori_loop(0, nchunk, chunk, None)
        plsc.subcore_barrier()

        nwo = n // 1000

        @pl.when(sid < nwo)
        def _():
            pltpu.sync_copy(acc_sh.at[pl.ds(sid * 1000, 1000)], ini_v.at[pl.ds(0, 1000)])
            pltpu.sync_copy(
                ini_v.at[pl.ds(0, 1000)], out_hbm.at[pl.ds(cid * n + sid * 1000, 1000)]
            )

    return deg_k(dst)


_KCH = 400


@functools.partial(jax.jit, static_argnums=(3, 4))
def _msg_pass(src, dst3, y, n, w):
    e = src.shape[0]
    epw = e // NW
    K = _KCH
    nchunk = epw // K

    @functools.partial(
        pl.kernel,
        out_type=jax.ShapeDtypeStruct((NC, n, w), F32),
        mesh=_sc_mesh(),
        scratch_types=[
            pltpu.VMEM((epw,), jnp.int32),
            pltpu.VMEM((nchunk, K), jnp.int32),
            pltpu.VMEM((2, K, w), F32),
            pltpu.VMEM_SHARED((n, w), F32),
            pltpu.SemaphoreType.DMA,
            pltpu.SemaphoreType.DMA,
        ],
        compiler_params=pltpu.CompilerParams(use_tc_tiling_on_sc=False),
    )
    def msg_k(src_hbm, dst_hbm, y_hbm, out_hbm, sidx_v, didx_v, rows_v, acc_sh, sem0, sem1):
        cid = lax.axis_index("c")
        sid = lax.axis_index("s")
        wid = sid * NC + cid
        sems = [sem0, sem1]

        zvec = jnp.zeros((L,), F32)

        @pl.when(sid < n // 1000)
        def _():
            def zb(r, _):
                for j in range(w // L):
                    rows_v[0, r, pl.ds(j * L, L)] = zvec
                return _

            lax.fori_loop(0, K, zb, None)
            for off, sz in ((0, K), (K, K), (2 * K, 1000 - 2 * K)):
                pltpu.sync_copy(
                    rows_v.at[0, pl.ds(0, sz)],
                    acc_sh.at[pl.ds(sid * 1000 + off, sz)],
                )

        plsc.subcore_barrier()

        base = wid * epw
        pltpu.sync_copy(src_hbm.at[pl.ds(base, epw)], sidx_v)
        pltpu.sync_copy(dst_hbm.at[wid], didx_v)

        g = [None, None]
        g[0] = pltpu.async_copy(y_hbm.at[sidx_v.at[pl.ds(0, K)]], rows_v.at[0], sem0)
        for i in range(nchunk):
            s = i % 2
            o = 1 - s
            if i + 1 < nchunk:
                g[o] = pltpu.async_copy(
                    y_hbm.at[sidx_v.at[pl.ds((i + 1) * K, K)]], rows_v.at[o], sems[o]
                )
            g[s].wait()
            pltpu.sync_copy(rows_v.at[s], acc_sh.at[didx_v.at[i]], add=True)

        plsc.subcore_barrier()

        @pl.when(sid < n // 1000)
        def _():
            for off, sz in ((0, K), (K, K), (2 * K, 1000 - 2 * K)):
                pltpu.sync_copy(
                    acc_sh.at[pl.ds(sid * 1000 + off, sz)],
                    rows_v.at[0, pl.ds(0, sz)],
                )
                pltpu.sync_copy(
                    rows_v.at[0, pl.ds(0, sz)],
                    out_hbm.at[cid, pl.ds(sid * 1000 + off, sz)],
                )

    return msg_k(src, dst3, y)


def _two_halves_specs(n, w):
    return [
        pl.BlockSpec((1, _BLK, w), lambda i: (0, i, 0)),
        pl.BlockSpec((1, _BLK, w), lambda i: (1, i, 0)),
    ]


_BLK = 1000


def _stage1_body(deg0_ref, deg1_ref, x_ref, w1_ref, y1_ref, dinv_ref):
    deg = deg0_ref[...] + deg1_ref[...]
    dinv = lax.rsqrt(deg)
    xw = jnp.dot(x_ref[...], w1_ref[...], preferred_element_type=F32)
    y1_ref[...] = xw * dinv
    dinv_ref[...] = dinv


@jax.jit
def _tc_stage1(deg_flat, x, w1):
    n, d_in = x.shape
    d_hid = w1.shape[1]
    nb = n // _BLK
    degf = deg_flat.reshape(2 * n, 1)
    return pl.pallas_call(
        _stage1_body,
        grid=(nb,),
        in_specs=[
            pl.BlockSpec((_BLK, 1), lambda i: (i, 0)),
            pl.BlockSpec((_BLK, 1), lambda i, nb=nb: (i + nb, 0)),
            pl.BlockSpec((_BLK, d_in), lambda i: (i, 0)),
            pl.BlockSpec((d_in, d_hid), lambda i: (0, 0)),
        ],
        out_specs=[
            pl.BlockSpec((_BLK, d_hid), lambda i: (i, 0)),
            pl.BlockSpec((_BLK, 1), lambda i: (i, 0)),
        ],
        out_shape=[
            jax.ShapeDtypeStruct((n, d_hid), F32),
            jax.ShapeDtypeStruct((n, 1), F32),
        ],
    )(degf, degf, x, w1)


def _stage2_body(a0_ref, a1_ref, y1_ref, dinv_ref, b1_ref, w2_ref, y2_ref):
    dinv = dinv_ref[...]
    h = (a0_ref[0] + a1_ref[0] + y1_ref[...]) * dinv + b1_ref[...]
    h = jnp.maximum(h, 0.0)
    y2_ref[...] = jnp.dot(h, w2_ref[...], preferred_element_type=F32) * dinv


@jax.jit
def _tc_stage2(p_parts, y1, dinv, b1r, w2p):
    n, d_hid = y1.shape
    wpad = w2p.shape[1]
    return pl.pallas_call(
        _stage2_body,
        grid=(n // _BLK,),
        in_specs=_two_halves_specs(n, d_hid) + [
            pl.BlockSpec((_BLK, d_hid), lambda i: (i, 0)),
            pl.BlockSpec((_BLK, 1), lambda i: (i, 0)),
            pl.BlockSpec((1, d_hid), lambda i: (0, 0)),
            pl.BlockSpec((d_hid, wpad), lambda i: (0, 0)),
        ],
        out_specs=pl.BlockSpec((_BLK, wpad), lambda i: (i, 0)),
        out_shape=jax.ShapeDtypeStruct((n, wpad), F32),
    )(p_parts, p_parts, y1, dinv, b1r, w2p)


def _stage3_body(q0_ref, q1_ref, y2_ref, dinv_ref, b2_ref, out_ref):
    out_ref[...] = (
        (q0_ref[0] + q1_ref[0] + y2_ref[...]) * dinv_ref[...] + b2_ref[...]
    )


@jax.jit
def _tc_stage3(q_parts, y2p, dinv, b2p):
    n, wpad = y2p.shape
    return pl.pallas_call(
        _stage3_body,
        grid=(n // _BLK,),
        in_specs=_two_halves_specs(n, wpad) + [
            pl.BlockSpec((_BLK, wpad), lambda i: (i, 0)),
            pl.BlockSpec((_BLK, 1), lambda i: (i, 0)),
            pl.BlockSpec((1, wpad), lambda i: (0, 0)),
        ],
        out_specs=pl.BlockSpec((_BLK, wpad), lambda i: (i, 0)),
        out_shape=jax.ShapeDtypeStruct((n, wpad), F32),
    )(q_parts, q_parts, y2p, dinv, b2p)


def kernel(x, edge_index, W1, b1, W2, b2):
    n, _ = x.shape
    d_hid = W1.shape[1]
    d_out = W2.shape[1]
    wpad = 16

    src = edge_index[0]
    dst = edge_index[1]
    e = src.shape[0]
    dst3 = dst.reshape(NW, e // (NW * _KCH), _KCH)

    deg_flat = _deg_pass(dst, n)
    y1, dinv = _tc_stage1(deg_flat, x, W1)
    p_parts = _msg_pass(src, dst3, y1, n, d_hid)

    b1r = b1.reshape(1, d_hid)
    w2p = jnp.zeros((d_hid, wpad), F32).at[:, :d_out].set(W2)
    b2p = jnp.zeros((1, wpad), F32).at[0, :d_out].set(b2)

    y2p = _tc_stage2(p_parts, y1, dinv, b1r, w2p)
    q_parts = _msg_pass(src, dst3, y2p, n, wpad)
    out = _tc_stage3(q_parts, y2p, dinv, b2p)
    return out[:, :d_out]


# speedup vs baseline: 52.3667x; 1.0915x over previous
import functools

import jax
import jax.numpy as jnp
from jax import lax
from jax.experimental import pallas as pl
from jax.experimental.pallas import tpu as pltpu
from jax.experimental.pallas import tpu_sc as plsc

F32 = jnp.float32
NC = 2
NS = 16
L = 16
NW = NC * NS


def _sc_mesh():
    return plsc.VectorSubcoreMesh(
        core_axis_name="c", subcore_axis_name="s", num_cores=NC, num_subcores=NS
    )


def _fill_1d(ref, total, val):
    vec = jnp.full((L,), val, F32)

    def body(i, _):
        ref[pl.ds(i * L, L)] = vec
        return _

    lax.fori_loop(0, total // L, body, None)


@functools.partial(jax.jit, static_argnums=(1,))
def _deg_pass(ei, n):
    e = ei.shape[1]
    epw = e // NW
    K = 2000
    nchunk = epw // K

    @functools.partial(
        pl.kernel,
        out_type=jax.ShapeDtypeStruct((NC * n,), F32),
        mesh=_sc_mesh(),
        scratch_types=[
            pltpu.VMEM((K,), jnp.int32),
            pltpu.VMEM((K,), F32),
            pltpu.VMEM((K,), F32),
            pltpu.VMEM_SHARED((n,), F32),
        ],
        compiler_params=pltpu.CompilerParams(use_tc_tiling_on_sc=False),
    )
    def deg_k(ei_hbm, out_hbm, idx_v, ones_v, ini_v, acc_sh):
        cid = lax.axis_index("c")
        sid = lax.axis_index("s")
        wid = sid * NC + cid

        _fill_1d(ones_v, K, 1.0)

        @pl.when(sid == 0)
        def _():
            initval = jnp.where(cid == 0, 1.0, 0.0).astype(F32)
            _fill_1d(ini_v, K, initval)

            def cp(i, _):
                pltpu.sync_copy(ini_v, acc_sh.at[pl.ds(i * K, K)])
                return _

            lax.fori_loop(0, n // K, cp, None)

        plsc.subcore_barrier()

        base = wid * epw

        def chunk(i, _):
            pltpu.sync_copy(ei_hbm.at[1, pl.ds(base + i * K, K)], idx_v)
            pltpu.sync_copy(ones_v, acc_sh.at[idx_v], add=True)
            return _

        lax.fori_loop(0, nchunk, chunk, None)
        plsc.subcore_barrier()

        nwo = n // 1000

        @pl.when(sid < nwo)
        def _():
            pltpu.sync_copy(acc_sh.at[pl.ds(sid * 1000, 1000)], ini_v.at[pl.ds(0, 1000)])
            pltpu.sync_copy(
                ini_v.at[pl.ds(0, 1000)], out_hbm.at[pl.ds(cid * n + sid * 1000, 1000)]
            )

    return deg_k(ei)


_KCH = 400


@functools.partial(jax.jit, static_argnums=(2, 3))
def _msg_pass(ei, y, n, w):
    e = ei.shape[1]
    epw = e // NW
    K = _KCH
    nchunk = epw // K

    @functools.partial(
        pl.kernel,
        out_type=jax.ShapeDtypeStruct((NC, n, w), F32),
        mesh=_sc_mesh(),
        scratch_types=[
            pltpu.VMEM((epw,), jnp.int32),
            pltpu.VMEM((nchunk, K), jnp.int32),
            pltpu.VMEM((2, K, w), F32),
            pltpu.VMEM_SHARED((n, w), F32),
            pltpu.SemaphoreType.DMA,
            pltpu.SemaphoreType.DMA,
        ],
        compiler_params=pltpu.CompilerParams(use_tc_tiling_on_sc=False),
    )
    def msg_k(ei_hbm, y_hbm, out_hbm, sidx_v, didx_v, rows_v, acc_sh, sem0, sem1):
        cid = lax.axis_index("c")
        sid = lax.axis_index("s")
        wid = sid * NC + cid
        sems = [sem0, sem1]

        zvec = jnp.zeros((L,), F32)

        @pl.when(sid < n // 1000)
        def _():
            def zb(r, _):
                for j in range(w // L):
                    rows_v[0, r, pl.ds(j * L, L)] = zvec
                return _

            lax.fori_loop(0, K, zb, None)
            for off, sz in ((0, K), (K, K), (2 * K, 1000 - 2 * K)):
                pltpu.sync_copy(
                    rows_v.at[0, pl.ds(0, sz)],
                    acc_sh.at[pl.ds(sid * 1000 + off, sz)],
                )

        plsc.subcore_barrier()

        base = wid * epw
        dcps = [
            pltpu.async_copy(
                ei_hbm.at[1, pl.ds(base + i * K, K)], didx_v.at[i], sem1
            )
            for i in range(nchunk)
        ]
        pltpu.sync_copy(ei_hbm.at[0, pl.ds(base, epw)], sidx_v)

        g = [None, None]
        g[0] = pltpu.async_copy(y_hbm.at[sidx_v.at[pl.ds(0, K)]], rows_v.at[0], sem0)
        for d in dcps:
            d.wait()
        for i in range(nchunk):
            s = i % 2
            o = 1 - s
            if i + 1 < nchunk:
                g[o] = pltpu.async_copy(
                    y_hbm.at[sidx_v.at[pl.ds((i + 1) * K, K)]], rows_v.at[o], sems[o]
                )
            g[s].wait()
            pltpu.sync_copy(rows_v.at[s], acc_sh.at[didx_v.at[i]], add=True)

        plsc.subcore_barrier()

        @pl.when(sid < n // 1000)
        def _():
            for off, sz in ((0, K), (K, K), (2 * K, 1000 - 2 * K)):
                pltpu.sync_copy(
                    acc_sh.at[pl.ds(sid * 1000 + off, sz)],
                    rows_v.at[0, pl.ds(0, sz)],
                )
                pltpu.sync_copy(
                    rows_v.at[0, pl.ds(0, sz)],
                    out_hbm.at[cid, pl.ds(sid * 1000 + off, sz)],
                )

    return msg_k(ei, y)


def _two_halves_specs(n, w):
    return [
        pl.BlockSpec((1, _BLK, w), lambda i: (0, i, 0)),
        pl.BlockSpec((1, _BLK, w), lambda i: (1, i, 0)),
    ]


_BLK = 2000


def _stage1_body(deg0_ref, deg1_ref, x_ref, w1_ref, y1_ref, dinv_ref):
    deg = deg0_ref[...] + deg1_ref[...]
    dinv = lax.rsqrt(deg)
    xw = jnp.dot(x_ref[...], w1_ref[...], preferred_element_type=F32)
    y1_ref[...] = xw * dinv
    dinv_ref[...] = dinv


@jax.jit
def _tc_stage1(deg_flat, x, w1):
    n, d_in = x.shape
    d_hid = w1.shape[1]
    nb = n // _BLK
    degf = deg_flat.reshape(2 * n, 1)
    return pl.pallas_call(
        _stage1_body,
        grid=(nb,),
        in_specs=[
            pl.BlockSpec((_BLK, 1), lambda i: (i, 0)),
            pl.BlockSpec((_BLK, 1), lambda i, nb=nb: (i + nb, 0)),
            pl.BlockSpec((_BLK, d_in), lambda i: (i, 0)),
            pl.BlockSpec((d_in, d_hid), lambda i: (0, 0)),
        ],
        out_specs=[
            pl.BlockSpec((_BLK, d_hid), lambda i: (i, 0)),
            pl.BlockSpec((_BLK, 1), lambda i: (i, 0)),
        ],
        out_shape=[
            jax.ShapeDtypeStruct((n, d_hid), F32),
            jax.ShapeDtypeStruct((n, 1), F32),
        ],
    )(degf, degf, x, w1)


def _stage2_body(a0_ref, a1_ref, y1_ref, dinv_ref, b1_ref, w2_ref, y2_ref):
    dinv = dinv_ref[...]
    h = (a0_ref[0] + a1_ref[0] + y1_ref[...]) * dinv + b1_ref[...]
    h = jnp.maximum(h, 0.0)
    y2_ref[...] = jnp.dot(h, w2_ref[...], preferred_element_type=F32) * dinv


@jax.jit
def _tc_stage2(p_parts, y1, dinv, b1r, w2p):
    n, d_hid = y1.shape
    wpad = w2p.shape[1]
    return pl.pallas_call(
        _stage2_body,
        grid=(n // _BLK,),
        in_specs=_two_halves_specs(n, d_hid) + [
            pl.BlockSpec((_BLK, d_hid), lambda i: (i, 0)),
            pl.BlockSpec((_BLK, 1), lambda i: (i, 0)),
            pl.BlockSpec((1, d_hid), lambda i: (0, 0)),
            pl.BlockSpec((d_hid, wpad), lambda i: (0, 0)),
        ],
        out_specs=pl.BlockSpec((_BLK, wpad), lambda i: (i, 0)),
        out_shape=jax.ShapeDtypeStruct((n, wpad), F32),
    )(p_parts, p_parts, y1, dinv, b1r, w2p)


def _stage3_body(q0_ref, q1_ref, y2_ref, dinv_ref, b2_ref, out_ref):
    full = (q0_ref[0] + q1_ref[0] + y2_ref[...]) * dinv_ref[...] + b2_ref[...]
    out_ref[...] = full[:, : out_ref.shape[1]]


@functools.partial(jax.jit, static_argnums=(4,))
def _tc_stage3(q_parts, y2p, dinv, b2p, d_out):
    n, wpad = y2p.shape
    return pl.pallas_call(
        _stage3_body,
        grid=(n // _BLK,),
        in_specs=_two_halves_specs(n, wpad) + [
            pl.BlockSpec((_BLK, wpad), lambda i: (i, 0)),
            pl.BlockSpec((_BLK, 1), lambda i: (i, 0)),
            pl.BlockSpec((1, wpad), lambda i: (0, 0)),
        ],
        out_specs=pl.BlockSpec((_BLK, d_out), lambda i: (i, 0)),
        out_shape=jax.ShapeDtypeStruct((n, d_out), F32),
    )(q_parts, q_parts, y2p, dinv, b2p)


def kernel(x, edge_index, W1, b1, W2, b2):
    n, _ = x.shape
    d_hid = W1.shape[1]
    d_out = W2.shape[1]
    wpad = 16

    deg_flat = _deg_pass(edge_index, n)
    y1, dinv = _tc_stage1(deg_flat, x, W1)
    p_parts = _msg_pass(edge_index, y1, n, d_hid)

    b1r = b1.reshape(1, d_hid)
    w2p = jnp.zeros((d_hid, wpad), F32).at[:, :d_out].set(W2)
    b2p = jnp.zeros((1, wpad), F32).at[0, :d_out].set(b2)

    y2p = _tc_stage2(p_parts, y1, dinv, b1r, w2p)
    q_parts = _msg_pass(edge_index, y2p, n, wpad)
    return _tc_stage3(q_parts, y2p, dinv, b2p, d_out)


# speedup vs baseline: 52.9356x; 1.0109x over previous
import functools

import jax
import jax.numpy as jnp
from jax import lax
from jax.experimental import pallas as pl
from jax.experimental.pallas import tpu as pltpu
from jax.experimental.pallas import tpu_sc as plsc

F32 = jnp.float32
NC = 2
NS = 16
L = 16
NW = NC * NS


def _sc_mesh():
    return plsc.VectorSubcoreMesh(
        core_axis_name="c", subcore_axis_name="s", num_cores=NC, num_subcores=NS
    )


def _fill_1d(ref, total, val):
    vec = jnp.full((L,), val, F32)

    def body(i, _):
        ref[pl.ds(i * L, L)] = vec
        return _

    lax.fori_loop(0, total // L, body, None)


@functools.partial(jax.jit, static_argnums=(1,))
def _deg_pass(ei, n):
    e = ei.shape[1]
    epw = e // NW
    K = 2000
    nchunk = epw // K

    @functools.partial(
        pl.kernel,
        out_type=jax.ShapeDtypeStruct((NC * n,), F32),
        mesh=_sc_mesh(),
        scratch_types=[
            pltpu.VMEM((K,), jnp.int32),
            pltpu.VMEM((K,), F32),
            pltpu.VMEM((K,), F32),
            pltpu.VMEM_SHARED((n,), F32),
        ],
        compiler_params=pltpu.CompilerParams(use_tc_tiling_on_sc=False),
    )
    def deg_k(ei_hbm, out_hbm, idx_v, ones_v, ini_v, acc_sh):
        cid = lax.axis_index("c")
        sid = lax.axis_index("s")
        wid = sid * NC + cid

        _fill_1d(ones_v, K, 1.0)

        @pl.when(sid == 0)
        def _():
            initval = jnp.where(cid == 0, 1.0, 0.0).astype(F32)
            _fill_1d(ini_v, K, initval)

            def cp(i, _):
                pltpu.sync_copy(ini_v, acc_sh.at[pl.ds(i * K, K)])
                return _

            lax.fori_loop(0, n // K, cp, None)

        plsc.subcore_barrier()

        base = wid * epw

        def chunk(i, _):
            pltpu.sync_copy(ei_hbm.at[1, pl.ds(base + i * K, K)], idx_v)
            pltpu.sync_copy(ones_v, acc_sh.at[idx_v], add=True)
            return _

        lax.fori_loop(0, nchunk, chunk, None)
        plsc.subcore_barrier()

        nwo = n // 1000

        @pl.when(sid < nwo)
        def _():
            pltpu.sync_copy(acc_sh.at[pl.ds(sid * 1000, 1000)], ini_v.at[pl.ds(0, 1000)])
            pltpu.sync_copy(
                ini_v.at[pl.ds(0, 1000)], out_hbm.at[pl.ds(cid * n + sid * 1000, 1000)]
            )

    return deg_k(ei)


_KCH = 400


@functools.partial(jax.jit, static_argnums=(2, 3))
def _msg_pass(ei, y, n, w):
    e = ei.shape[1]
    epw = e // NW
    K = _KCH
    nchunk = epw // K

    @functools.partial(
        pl.kernel,
        out_type=jax.ShapeDtypeStruct((NC, n, w), F32),
        mesh=_sc_mesh(),
        scratch_types=[
            pltpu.VMEM((epw,), jnp.int32),
            pltpu.VMEM((nchunk, K), jnp.int32),
            pltpu.VMEM((2, K, w), F32),
            pltpu.VMEM_SHARED((n, w), F32),
            pltpu.SemaphoreType.DMA,
            pltpu.SemaphoreType.DMA,
        ],
        compiler_params=pltpu.CompilerParams(use_tc_tiling_on_sc=False),
    )
    def msg_k(ei_hbm, y_hbm, out_hbm, sidx_v, didx_v, rows_v, acc_sh, sem0, sem1):
        cid = lax.axis_index("c")
        sid = lax.axis_index("s")
        wid = sid * NC + cid
        sems = [sem0, sem1]

        zvec = jnp.zeros((L,), F32)

        @pl.when(sid < n // 1000)
        def _():
            def zb(r, _):
                for j in range(w // L):
                    rows_v[0, r, pl.ds(j * L, L)] = zvec
                return _

            lax.fori_loop(0, K, zb, None)
            for off, sz in ((0, K), (K, K), (2 * K, 1000 - 2 * K)):
                pltpu.sync_copy(
                    rows_v.at[0, pl.ds(0, sz)],
                    acc_sh.at[pl.ds(sid * 1000 + off, sz)],
                )

        plsc.subcore_barrier()

        base = wid * epw
        dcps = [
            pltpu.async_copy(
                ei_hbm.at[1, pl.ds(base + i * K, K)], didx_v.at[i], sem1
            )
            for i in range(nchunk)
        ]
        pltpu.sync_copy(ei_hbm.at[0, pl.ds(base, epw)], sidx_v)

        g = [None, None]
        g[0] = pltpu.async_copy(y_hbm.at[sidx_v.at[pl.ds(0, K)]], rows_v.at[0], sem0)
        for d in dcps:
            d.wait()
        for i in range(nchunk):
            s = i % 2
            o = 1 - s
            if i + 1 < nchunk:
                g[o] = pltpu.async_copy(
                    y_hbm.at[sidx_v.at[pl.ds((i + 1) * K, K)]], rows_v.at[o], sems[o]
                )
            g[s].wait()
            pltpu.sync_copy(rows_v.at[s], acc_sh.at[didx_v.at[i]], add=True)

        plsc.subcore_barrier()

        @pl.when(sid < n // 1000)
        def _():
            for off, sz in ((0, K), (K, K), (2 * K, 1000 - 2 * K)):
                pltpu.sync_copy(
                    acc_sh.at[pl.ds(sid * 1000 + off, sz)],
                    rows_v.at[0, pl.ds(0, sz)],
                )
                pltpu.sync_copy(
                    rows_v.at[0, pl.ds(0, sz)],
                    out_hbm.at[cid, pl.ds(sid * 1000 + off, sz)],
                )

    return msg_k(ei, y)


def _two_halves_specs(n, w):
    return [
        pl.BlockSpec((1, _BLK, w), lambda i: (0, i, 0)),
        pl.BlockSpec((1, _BLK, w), lambda i: (1, i, 0)),
    ]


_BLK = 2000


def _mm_body(a_ref, b_ref, o_ref):
    o_ref[...] = jnp.dot(a_ref[...], b_ref[...], preferred_element_type=F32)


@jax.jit
def _tc_matmul(a, b):
    n, k = a.shape
    m = b.shape[1]
    return pl.pallas_call(
        _mm_body,
        grid=(n // _BLK,),
        in_specs=[
            pl.BlockSpec((_BLK, k), lambda i: (i, 0)),
            pl.BlockSpec((k, m), lambda i: (0, 0)),
        ],
        out_specs=pl.BlockSpec((_BLK, m), lambda i: (i, 0)),
        out_shape=jax.ShapeDtypeStruct((n, m), F32),
    )(a, b)


def kernel(x, edge_index, W1, b1, W2, b2):
    n, _ = x.shape
    d_hid = W1.shape[1]
    d_out = W2.shape[1]
    wpad = 16

    deg_flat = _deg_pass(edge_index, n)
    dinv = lax.rsqrt(deg_flat[:n] + deg_flat[n:])[:, None]

    xw = _tc_matmul(x, W1)
    y1 = xw * dinv
    p = _msg_pass(edge_index, y1, n, d_hid)

    h = jnp.maximum((p[0] + p[1] + y1) * dinv + b1[None, :], 0.0)
    w2p = jnp.zeros((d_hid, wpad), F32).at[:, :d_out].set(W2)
    hw = _tc_matmul(h, w2p)
    y2p = hw * dinv
    q = _msg_pass(edge_index, y2p, n, wpad)

    out = (q[0, :, :d_out] + q[1, :, :d_out] + y2p[:, :d_out]) * dinv + b2[None, :]
    return out
